# baseline (speedup 1.0000x reference)
import functools

import jax
import jax.numpy as jnp
from jax import lax
from jax.experimental import pallas as pl
from jax.experimental.pallas import tpu as pltpu
from jax.experimental.pallas import tpu_sc as plsc

N_U = 10000
N_I = 10000
E = 320000
D = 128
R = 5
L = 16
NW = 32
EPW = E // NW
C = 80
NCH = EPW // C
NJ = D // L
PD = 384


def _matmul_body(x_ref, p_ref, o_ref):
    o_ref[...] = jnp.dot(x_ref[...], p_ref[...],
                         preferred_element_type=jnp.float32
                         ).astype(jnp.bfloat16)


def _dense_transform(ufeat, Pall):
    BLK = 1000
    return pl.pallas_call(
        _matmul_body,
        grid=(N_U // BLK,),
        in_specs=[
            pl.BlockSpec((BLK, D), lambda i: (i, 0)),
            pl.BlockSpec((D, R * D), lambda i: (0, 0)),
        ],
        out_specs=pl.BlockSpec((BLK, R * D), lambda i: (i, 0)),
        out_shape=jax.ShapeDtypeStruct((N_U, R * D), jnp.bfloat16),
    )(ufeat, Pall)


_MESH = plsc.VectorSubcoreMesh(core_axis_name="c", subcore_axis_name="s")


@functools.partial(
    pl.kernel,
    out_type=jax.ShapeDtypeStruct((E,), jnp.float32),
    mesh=_MESH,
    compiler_params=pltpu.CompilerParams(needs_layout_passes=False),
    scratch_types=[
        pltpu.VMEM((C,), jnp.int32),
        pltpu.VMEM((C,), jnp.int32),
        pltpu.VMEM((C,), jnp.int32),
        pltpu.VMEM((C,), jnp.int32),
        pltpu.VMEM((C, PD), jnp.float32),
        pltpu.VMEM((C, PD), jnp.float32),
        pltpu.VMEM((C, D), jnp.float32),
        pltpu.VMEM((C, D), jnp.float32),
        pltpu.VMEM((R * C,), jnp.float32),
        pltpu.VMEM((C,), jnp.float32),
        pltpu.VMEM((C,), jnp.float32),
        pltpu.SemaphoreType.DMA,
        pltpu.SemaphoreType.DMA,
        pltpu.SemaphoreType.DMA,
        pltpu.SemaphoreType.DMA,
        pltpu.SemaphoreType.DMA,
        pltpu.SemaphoreType.DMA,
    ],
)
def _sc_edges(hu_hbm, if_hbm, src_hbm, dst_hbm, out_hbm,
              sidxA, didxA, sidxB, didxB, huA, huB, vA, vB,
              scores, outcA, outcB, gsemA, gsemB,
              isemA, isemB, osemA, osemB):
    wid = lax.axis_index("s") * 2 + lax.axis_index("c")
    base = wid * EPW
    iota = lax.iota(jnp.int32, L)
    lane15 = iota == (L - 1)
    cbase = [iota + (r * C - (L - 1)) for r in range(R)]

    bufA = (sidxA, didxA, huA, vA, gsemA, isemA, outcA, osemA)
    bufB = (sidxB, didxB, huB, vB, gsemB, isemB, outcB, osemB)

    def copy_idx(k, buf):
        off = base + k * C
        pltpu.async_copy(src_hbm.at[pl.ds(off, C)], buf[0], buf[5])
        pltpu.async_copy(dst_hbm.at[pl.ds(off, C)], buf[1], buf[5])

    def wait_idx(buf):
        pltpu.make_async_copy(src_hbm.at[pl.ds(0, C)], buf[0], buf[5]).wait()
        pltpu.make_async_copy(dst_hbm.at[pl.ds(0, C)], buf[1], buf[5]).wait()

    def issue_gather(buf):
        pltpu.async_copy(hu_hbm.at[buf[0]], buf[2], buf[4])
        pltpu.async_copy(if_hbm.at[buf[1]], buf[3], buf[4])

    def wait_gather(buf):
        pltpu.make_async_copy(hu_hbm.at[buf[0]], buf[2], buf[4]).wait()
        pltpu.make_async_copy(if_hbm.at[buf[1]], buf[3], buf[4]).wait()

    def wait_out(buf):
        pltpu.make_async_copy(buf[6], out_hbm.at[pl.ds(0, C)], buf[7]).wait()

    def compute(k, buf):
        hurows, vrows = buf[2], buf[3]
        outc = buf[6]

        @plsc.parallel_loop(0, C, 1, unroll=4)
        def edge_body(i):
            v = [vrows[i, pl.ds(L * j, L)] for j in range(NJ)]
            for r in range(R):
                h = []
                for j in range(NJ // 2):
                    w = hurows[i, pl.ds((r * D + 2 * L * j) // 2, L)]
                    a, b = plsc.unpack(plsc.bitcast(w, jnp.bfloat16),
                                       format=plsc.PackFormat.INTERLEAVED)
                    h.append(a)
                    h.append(b)
                acc = h[0] * v[0]
                for j in range(1, NJ):
                    acc = acc + h[j] * v[j]
                s = plsc.cumsum(acc)
                plsc.store_scatter(scores, [cbase[r] + i], s, mask=lane15)

        for g in range(C // L):
            s = [scores[pl.ds(r * C + g * L, L)] for r in range(R)]
            m = s[0]
            for r in range(1, R):
                m = jnp.maximum(m, s[r])
            e = [jnp.exp(sr - m) for sr in s]
            den = e[0]
            num = e[0]
            for r in range(1, R):
                den = den + e[r]
                num = num + e[r] * float(r + 1)
            outc[pl.ds(g * L, L)] = num / den

        pltpu.async_copy(outc, out_hbm.at[pl.ds(base + k * C, C)], buf[7])

    copy_idx(0, bufA)
    pltpu.async_copy(out_hbm.at[pl.ds(0, C)], outcA, osemA)
    pltpu.async_copy(out_hbm.at[pl.ds(0, C)], outcB, osemB)
    wait_idx(bufA)
    issue_gather(bufA)
    copy_idx(1, bufB)

    def pair_body(m, carry):
        k0 = 2 * m
        wait_idx(bufB)
        issue_gather(bufB)
        wait_gather(bufA)
        copy_idx(k0 + 2, bufA)
        wait_out(bufA)
        compute(k0, bufA)

        wait_idx(bufA)
        issue_gather(bufA)
        wait_gather(bufB)

        @pl.when(k0 + 3 < NCH)
        def _():
            copy_idx(k0 + 3, bufB)

        wait_out(bufB)
        compute(k0 + 1, bufB)
        return carry

    lax.fori_loop(0, NCH // 2, pair_body, 0)

    wait_gather(bufA)
    wait_out(bufA)
    compute(NCH - 1, bufA)
    wait_out(bufB)
    wait_out(bufA)


def kernel(ufeat, ifeat, edge_index, Ps):
    src = edge_index[0].astype(jnp.int32)
    dst = edge_index[1].astype(jnp.int32)
    Pall = jnp.transpose(Ps, (1, 0, 2)).reshape(D, R * D)
    Pall = Pall.reshape(D, R, D // 32, 2, L).transpose(0, 1, 2, 4, 3)
    Pall = Pall.reshape(D, R * D)
    hu = _dense_transform(ufeat, Pall)
    hu_packed = lax.bitcast_convert_type(
        hu.reshape(N_U, R * D // 2, 2), jnp.float32)
    hu_packed = jnp.pad(hu_packed, ((0, 0), (0, PD - R * D // 2)))
    return _sc_edges(hu_packed, ifeat, src, dst)


# speedup vs baseline: 1.5257x; 1.5257x over previous
import functools

import jax
import jax.numpy as jnp
from jax import lax
from jax.experimental import pallas as pl
from jax.experimental.pallas import tpu as pltpu
from jax.experimental.pallas import tpu_sc as plsc

N_U = 10000
N_I = 10000
E = 320000
D = 128
R = 5
L = 16
NW = 32
EPW = E // NW
C = 80
NCH = EPW // C
NJ = D // L


def _matmul_body(x_ref, p_ref, o_ref):
    o_ref[...] = jnp.dot(x_ref[...], p_ref[...],
                         preferred_element_type=jnp.float32)


def _dense_transform(ufeat, Pall):
    BLK = 1000
    return pl.pallas_call(
        _matmul_body,
        grid=(N_U // BLK,),
        in_specs=[
            pl.BlockSpec((BLK, D), lambda i: (i, 0)),
            pl.BlockSpec((D, R * D), lambda i: (0, 0)),
        ],
        out_specs=pl.BlockSpec((BLK, R * D), lambda i: (i, 0)),
        out_shape=jax.ShapeDtypeStruct((N_U, R * D), jnp.float32),
    )(ufeat, Pall)


_MESH = plsc.VectorSubcoreMesh(core_axis_name="c", subcore_axis_name="s")


@functools.partial(
    pl.kernel,
    out_type=jax.ShapeDtypeStruct((E,), jnp.float32),
    mesh=_MESH,
    compiler_params=pltpu.CompilerParams(needs_layout_passes=False),
    scratch_types=[
        pltpu.VMEM((C,), jnp.int32),
        pltpu.VMEM((C,), jnp.int32),
        pltpu.VMEM((C,), jnp.int32),
        pltpu.VMEM((C,), jnp.int32),
        pltpu.VMEM((C, R * D), jnp.float32),
        pltpu.VMEM((C, R * D), jnp.float32),
        pltpu.VMEM((C, D), jnp.float32),
        pltpu.VMEM((C, D), jnp.float32),
        pltpu.VMEM((R * C,), jnp.float32),
        pltpu.VMEM((C,), jnp.float32),
        pltpu.VMEM((C,), jnp.float32),
        pltpu.SemaphoreType.DMA,
        pltpu.SemaphoreType.DMA,
        pltpu.SemaphoreType.DMA,
        pltpu.SemaphoreType.DMA,
        pltpu.SemaphoreType.DMA,
        pltpu.SemaphoreType.DMA,
    ],
)
def _sc_edges(hu_hbm, if_hbm, src_hbm, dst_hbm, out_hbm,
              sidxA, didxA, sidxB, didxB, huA, huB, vA, vB,
              scores, outcA, outcB, gsemA, gsemB,
              isemA, isemB, osemA, osemB):
    wid = lax.axis_index("s") * 2 + lax.axis_index("c")
    base = wid * EPW
    iota = lax.iota(jnp.int32, L)
    lane15 = iota == (L - 1)
    cbase = [iota + (r * C - (L - 1)) for r in range(R)]

    bufA = (sidxA, didxA, huA, vA, gsemA, isemA, outcA, osemA)
    bufB = (sidxB, didxB, huB, vB, gsemB, isemB, outcB, osemB)

    def copy_idx(k, buf):
        off = base + k * C
        pltpu.async_copy(src_hbm.at[pl.ds(off, C)], buf[0], buf[5])
        pltpu.async_copy(dst_hbm.at[pl.ds(off, C)], buf[1], buf[5])

    def wait_idx(buf):
        pltpu.make_async_copy(src_hbm.at[pl.ds(0, C)], buf[0], buf[5]).wait()
        pltpu.make_async_copy(dst_hbm.at[pl.ds(0, C)], buf[1], buf[5]).wait()

    def issue_gather(buf):
        pltpu.async_copy(hu_hbm.at[buf[0]], buf[2], buf[4])
        pltpu.async_copy(if_hbm.at[buf[1]], buf[3], buf[4])

    def wait_gather(buf):
        pltpu.make_async_copy(hu_hbm.at[buf[0]], buf[2], buf[4]).wait()
        pltpu.make_async_copy(if_hbm.at[buf[1]], buf[3], buf[4]).wait()

    def wait_out(buf):
        pltpu.make_async_copy(buf[6], out_hbm.at[pl.ds(0, C)], buf[7]).wait()

    def compute(k, buf):
        hurows, vrows = buf[2], buf[3]
        outc = buf[6]

        @plsc.parallel_loop(0, C, 1, unroll=4)
        def edge_body(i):
            v = [vrows[i, pl.ds(L * j, L)] for j in range(NJ)]
            for r in range(R):
                acc = hurows[i, pl.ds(r * D, L)] * v[0]
                for j in range(1, NJ):
                    acc = acc + hurows[i, pl.ds(r * D + L * j, L)] * v[j]
                s = plsc.cumsum(acc)
                plsc.store_scatter(scores, [cbase[r] + i], s, mask=lane15)

        for g in range(C // L):
            s = [scores[pl.ds(r * C + g * L, L)] for r in range(R)]
            m = s[0]
            for r in range(1, R):
                m = jnp.maximum(m, s[r])
            e = [jnp.exp(sr - m) for sr in s]
            den = e[0]
            num = e[0]
            for r in range(1, R):
                den = den + e[r]
                num = num + e[r] * float(r + 1)
            outc[pl.ds(g * L, L)] = num / den

        pltpu.async_copy(outc, out_hbm.at[pl.ds(base + k * C, C)], buf[7])

    copy_idx(0, bufA)
    pltpu.async_copy(out_hbm.at[pl.ds(0, C)], outcA, osemA)
    pltpu.async_copy(out_hbm.at[pl.ds(0, C)], outcB, osemB)
    wait_idx(bufA)
    issue_gather(bufA)
    copy_idx(1, bufB)

    def pair_body(m, carry):
        k0 = 2 * m
        wait_idx(bufB)
        issue_gather(bufB)
        wait_gather(bufA)
        copy_idx(k0 + 2, bufA)
        wait_out(bufA)
        compute(k0, bufA)

        wait_idx(bufA)
        issue_gather(bufA)
        wait_gather(bufB)

        @pl.when(k0 + 3 < NCH)
        def _():
            copy_idx(k0 + 3, bufB)

        wait_out(bufB)
        compute(k0 + 1, bufB)
        return carry

    lax.fori_loop(0, NCH // 2, pair_body, 0)

    wait_gather(bufA)
    wait_out(bufA)
    compute(NCH - 1, bufA)
    wait_out(bufB)
    wait_out(bufA)


def kernel(ufeat, ifeat, edge_index, Ps):
    src = edge_index[0].astype(jnp.int32)
    dst = edge_index[1].astype(jnp.int32)
    Pall = jnp.transpose(Ps, (1, 0, 2)).reshape(D, R * D)
    hu = _dense_transform(ufeat, Pall)
    return _sc_edges(hu, ifeat, src, dst)


# speedup vs baseline: 1.8502x; 1.2126x over previous
import functools

import jax
import jax.numpy as jnp
from jax import lax
from jax.experimental import pallas as pl
from jax.experimental.pallas import tpu as pltpu
from jax.experimental.pallas import tpu_sc as plsc

N_U = 10000
N_I = 10000
E = 320000
D = 128
R = 5
L = 16
NW = 32
EPW = E // NW
C = 80
NCH = EPW // C
NJ = D // L


def _matmul_body(x_ref, p_ref, o_ref):
    o_ref[...] = jnp.dot(x_ref[...], p_ref[...],
                         preferred_element_type=jnp.float32)


def _dense_transform(ufeat, Pall):
    BLK = 1000
    return pl.pallas_call(
        _matmul_body,
        grid=(N_U // BLK,),
        in_specs=[
            pl.BlockSpec((BLK, D), lambda i: (i, 0)),
            pl.BlockSpec((D, R * D), lambda i: (0, 0)),
        ],
        out_specs=pl.BlockSpec((BLK, R * D), lambda i: (i, 0)),
        out_shape=jax.ShapeDtypeStruct((N_U, R * D), jnp.float32),
    )(ufeat, Pall)


_MESH = plsc.VectorSubcoreMesh(core_axis_name="c", subcore_axis_name="s")


@functools.partial(
    pl.kernel,
    out_type=jax.ShapeDtypeStruct((E,), jnp.float32),
    mesh=_MESH,
    compiler_params=pltpu.CompilerParams(needs_layout_passes=False),
    scratch_types=[
        pltpu.VMEM((C,), jnp.int32),
        pltpu.VMEM((C,), jnp.int32),
        pltpu.VMEM((C,), jnp.int32),
        pltpu.VMEM((C,), jnp.int32),
        pltpu.VMEM((C, R * D), jnp.float32),
        pltpu.VMEM((C, R * D), jnp.float32),
        pltpu.VMEM((C, D), jnp.float32),
        pltpu.VMEM((C, D), jnp.float32),
        pltpu.VMEM((R * C,), jnp.float32),
        pltpu.VMEM((C,), jnp.float32),
        pltpu.VMEM((C,), jnp.float32),
        pltpu.SemaphoreType.DMA,
        pltpu.SemaphoreType.DMA,
        pltpu.SemaphoreType.DMA,
        pltpu.SemaphoreType.DMA,
        pltpu.SemaphoreType.DMA,
        pltpu.SemaphoreType.DMA,
    ],
)
def _sc_edges(hu_hbm, if_hbm, src_hbm, dst_hbm, out_hbm,
              sidxA, didxA, sidxB, didxB, huA, huB, vA, vB,
              scores, outcA, outcB, gsemA, gsemB,
              isemA, isemB, osemA, osemB):
    wid = lax.axis_index("s") * 2 + lax.axis_index("c")
    base = wid * EPW
    iota = lax.iota(jnp.int32, L)
    lane15 = iota == (L - 1)
    cbase = [iota + (r * C - (L - 1)) for r in range(R)]

    bufA = (sidxA, didxA, huA, vA, gsemA, isemA, outcA, osemA)
    bufB = (sidxB, didxB, huB, vB, gsemB, isemB, outcB, osemB)

    def copy_idx(k, buf):
        off = base + k * C
        pltpu.async_copy(src_hbm.at[pl.ds(off, C)], buf[0], buf[5])
        pltpu.async_copy(dst_hbm.at[pl.ds(off, C)], buf[1], buf[5])

    def wait_idx(buf):
        pltpu.make_async_copy(src_hbm.at[pl.ds(0, C)], buf[0], buf[5]).wait()
        pltpu.make_async_copy(dst_hbm.at[pl.ds(0, C)], buf[1], buf[5]).wait()

    def issue_gather(buf):
        pltpu.async_copy(hu_hbm.at[buf[0]], buf[2], buf[4])
        pltpu.async_copy(if_hbm.at[buf[1]], buf[3], buf[4])

    def wait_gather(buf):
        pltpu.make_async_copy(hu_hbm.at[buf[0]], buf[2], buf[4]).wait()
        pltpu.make_async_copy(if_hbm.at[buf[1]], buf[3], buf[4]).wait()

    def wait_out(buf):
        pltpu.make_async_copy(buf[6], out_hbm.at[pl.ds(0, C)], buf[7]).wait()

    def compute(k, buf):
        hurows, vrows = buf[2], buf[3]
        outc = buf[6]

        @plsc.parallel_loop(0, C, 1, unroll=2)
        def edge_body(i):
            v = [vrows[i, pl.ds(L * j, L)] for j in range(NJ)]
            for r in range(R):
                acc = hurows[i, pl.ds(r * D, L)] * v[0]
                for j in range(1, NJ):
                    acc = acc + hurows[i, pl.ds(r * D + L * j, L)] * v[j]
                s = plsc.cumsum(acc)
                plsc.store_scatter(scores, [cbase[r] + i], s, mask=lane15)

        for g in range(C // L):
            s = [scores[pl.ds(r * C + g * L, L)] for r in range(R)]
            m = s[0]
            for r in range(1, R):
                m = jnp.maximum(m, s[r])
            e = [jnp.exp(sr - m) for sr in s]
            den = e[0]
            num = e[0]
            for r in range(1, R):
                den = den + e[r]
                num = num + e[r] * float(r + 1)
            outc[pl.ds(g * L, L)] = num / den

        pltpu.async_copy(outc, out_hbm.at[pl.ds(base + k * C, C)], buf[7])

    copy_idx(0, bufA)
    pltpu.async_copy(out_hbm.at[pl.ds(0, C)], outcA, osemA)
    pltpu.async_copy(out_hbm.at[pl.ds(0, C)], outcB, osemB)
    wait_idx(bufA)
    issue_gather(bufA)
    copy_idx(1, bufB)

    def pair_body(m, carry):
        k0 = 2 * m
        wait_idx(bufB)
        issue_gather(bufB)
        wait_gather(bufA)
        copy_idx(k0 + 2, bufA)
        wait_out(bufA)
        compute(k0, bufA)

        wait_idx(bufA)
        issue_gather(bufA)
        wait_gather(bufB)

        @pl.when(k0 + 3 < NCH)
        def _():
            copy_idx(k0 + 3, bufB)

        wait_out(bufB)
        compute(k0 + 1, bufB)
        return carry

    lax.fori_loop(0, NCH // 2, pair_body, 0)

    wait_gather(bufA)
    wait_out(bufA)
    compute(NCH - 1, bufA)
    wait_out(bufB)
    wait_out(bufA)


def kernel(ufeat, ifeat, edge_index, Ps):
    src = edge_index[0].astype(jnp.int32)
    dst = edge_index[1].astype(jnp.int32)
    Pall = jnp.transpose(Ps, (1, 0, 2)).reshape(D, R * D)
    hu = _dense_transform(ufeat, Pall)
    return _sc_edges(hu, ifeat, src, dst)


# speedup vs baseline: 2.0423x; 1.1038x over previous
import functools

import jax
import jax.numpy as jnp
from jax import lax
from jax.experimental import pallas as pl
from jax.experimental.pallas import tpu as pltpu
from jax.experimental.pallas import tpu_sc as plsc

N_U = 10000
N_I = 10000
E = 320000
D = 128
R = 5
L = 16
NW = 32
EPW = E // NW
C = 80
NCH = EPW // C
NJ = D // L


def _matmul_body(x_ref, p_ref, o_ref):
    o_ref[...] = jnp.dot(x_ref[...], p_ref[...],
                         preferred_element_type=jnp.float32)


def _dense_transform(ufeat, Pall):
    BLK = 1000
    return pl.pallas_call(
        _matmul_body,
        grid=(N_U // BLK,),
        in_specs=[
            pl.BlockSpec((BLK, D), lambda i: (i, 0)),
            pl.BlockSpec((D, R * D), lambda i: (0, 0)),
        ],
        out_specs=pl.BlockSpec((BLK, R * D), lambda i: (i, 0)),
        out_shape=jax.ShapeDtypeStruct((N_U, R * D), jnp.float32),
    )(ufeat, Pall)


_MESH = plsc.VectorSubcoreMesh(core_axis_name="c", subcore_axis_name="s")


@functools.partial(
    pl.kernel,
    out_type=jax.ShapeDtypeStruct((E,), jnp.float32),
    mesh=_MESH,
    compiler_params=pltpu.CompilerParams(needs_layout_passes=False),
    scratch_types=[
        pltpu.VMEM((C,), jnp.int32),
        pltpu.VMEM((C,), jnp.int32),
        pltpu.VMEM((C,), jnp.int32),
        pltpu.VMEM((C,), jnp.int32),
        pltpu.VMEM((C, R * D), jnp.float32),
        pltpu.VMEM((C, R * D), jnp.float32),
        pltpu.VMEM((C, D), jnp.float32),
        pltpu.VMEM((C, D), jnp.float32),
        pltpu.VMEM((R * C,), jnp.float32),
        pltpu.VMEM((C,), jnp.float32),
        pltpu.VMEM((C,), jnp.float32),
        pltpu.SemaphoreType.DMA,
        pltpu.SemaphoreType.DMA,
        pltpu.SemaphoreType.DMA,
        pltpu.SemaphoreType.DMA,
        pltpu.SemaphoreType.DMA,
        pltpu.SemaphoreType.DMA,
    ],
)
def _sc_edges(hu_hbm, if_hbm, src_hbm, dst_hbm, out_hbm,
              sidxA, didxA, sidxB, didxB, huA, huB, vA, vB,
              scores, outcA, outcB, gsemA, gsemB,
              isemA, isemB, osemA, osemB):
    wid = lax.axis_index("s") * 2 + lax.axis_index("c")
    base = wid * EPW
    iota = lax.iota(jnp.int32, L)
    lane15 = iota == (L - 1)
    cbase = [iota + (r * C - (L - 1)) for r in range(R)]

    bufA = (sidxA, didxA, huA, vA, gsemA, isemA, outcA, osemA)
    bufB = (sidxB, didxB, huB, vB, gsemB, isemB, outcB, osemB)

    def copy_idx(k, buf):
        off = base + k * C
        pltpu.async_copy(src_hbm.at[pl.ds(off, C)], buf[0], buf[5])
        pltpu.async_copy(dst_hbm.at[pl.ds(off, C)], buf[1], buf[5])

    def wait_idx(buf):
        pltpu.make_async_copy(src_hbm.at[pl.ds(0, C)], buf[0], buf[5]).wait()
        pltpu.make_async_copy(dst_hbm.at[pl.ds(0, C)], buf[1], buf[5]).wait()

    def issue_gather(buf):
        pltpu.async_copy(hu_hbm.at[buf[0]], buf[2], buf[4])
        pltpu.async_copy(if_hbm.at[buf[1]], buf[3], buf[4])

    def wait_gather(buf):
        pltpu.make_async_copy(hu_hbm.at[buf[0]], buf[2], buf[4]).wait()
        pltpu.make_async_copy(if_hbm.at[buf[1]], buf[3], buf[4]).wait()

    def wait_out(buf):
        pltpu.make_async_copy(buf[6], out_hbm.at[pl.ds(0, C)], buf[7]).wait()

    def compute(k, buf):
        hurows, vrows = buf[2], buf[3]
        outc = buf[6]

        @plsc.parallel_loop(0, C, 1, unroll=1)
        def edge_body(i):
            v = [vrows[i, pl.ds(L * j, L)] for j in range(NJ)]
            for r in range(R):
                acc = hurows[i, pl.ds(r * D, L)] * v[0]
                for j in range(1, NJ):
                    acc = acc + hurows[i, pl.ds(r * D + L * j, L)] * v[j]
                s = plsc.cumsum(acc)
                plsc.store_scatter(scores, [cbase[r] + i], s, mask=lane15)

        for g in range(C // L):
            s = [scores[pl.ds(r * C + g * L, L)] for r in range(R)]
            m = s[0]
            for r in range(1, R):
                m = jnp.maximum(m, s[r])
            e = [jnp.exp(sr - m) for sr in s]
            den = e[0]
            num = e[0]
            for r in range(1, R):
                den = den + e[r]
                num = num + e[r] * float(r + 1)
            outc[pl.ds(g * L, L)] = num / den

        pltpu.async_copy(outc, out_hbm.at[pl.ds(base + k * C, C)], buf[7])

    copy_idx(0, bufA)
    pltpu.async_copy(out_hbm.at[pl.ds(0, C)], outcA, osemA)
    pltpu.async_copy(out_hbm.at[pl.ds(0, C)], outcB, osemB)
    wait_idx(bufA)
    issue_gather(bufA)
    copy_idx(1, bufB)

    def pair_body(m, carry):
        k0 = 2 * m
        wait_idx(bufB)
        issue_gather(bufB)
        wait_gather(bufA)
        copy_idx(k0 + 2, bufA)
        wait_out(bufA)
        compute(k0, bufA)

        wait_idx(bufA)
        issue_gather(bufA)
        wait_gather(bufB)

        @pl.when(k0 + 3 < NCH)
        def _():
            copy_idx(k0 + 3, bufB)

        wait_out(bufB)
        compute(k0 + 1, bufB)
        return carry

    lax.fori_loop(0, NCH // 2, pair_body, 0)

    wait_gather(bufA)
    wait_out(bufA)
    compute(NCH - 1, bufA)
    wait_out(bufB)
    wait_out(bufA)


def kernel(ufeat, ifeat, edge_index, Ps):
    src = edge_index[0].astype(jnp.int32)
    dst = edge_index[1].astype(jnp.int32)
    Pall = jnp.transpose(Ps, (1, 0, 2)).reshape(D, R * D)
    hu = _dense_transform(ufeat, Pall)
    return _sc_edges(hu, ifeat, src, dst)
